# Initial kernel scaffold; baseline (speedup 1.0000x reference)
import jax
import jax.numpy as jnp
from jax.experimental import pallas as pl


def kernel(features, norm, edge_index, W0, W1, W2):
    raise NotImplementedError("write your pallas kernel here")


# speedup vs baseline: 3.3267x; 3.3267x over previous
import functools

import jax
import jax.numpy as jnp
from jax import lax
from jax.experimental import pallas as pl
from jax.experimental.pallas import tpu as pltpu
from jax.experimental.pallas import tpu_sc as plsc

_N = 10000
_E = 160000


def _tc_layer_body(x0_ref, x1_ref, norm_ref, w_ref, out0_ref, out1_ref,
                   *, relu_in, dh):
    x = jnp.concatenate([x0_ref[...], x1_ref[...]], axis=1)
    nrm = norm_ref[...]
    if relu_in:
        x = jnp.maximum(x * nrm, 0.0)
    y = jnp.dot(x, w_ref[...], preferred_element_type=jnp.float32)
    y = y * nrm
    out0_ref[...] = y[:, :dh]
    out1_ref[...] = y[:, dh:]


def _tc_layer(x0, x1, norm, w, relu_in):
    n, dhin = x0.shape
    dout = w.shape[1]
    dh = dout // 2
    blk = 1000
    return pl.pallas_call(
        functools.partial(_tc_layer_body, relu_in=relu_in, dh=dh),
        grid=(n // blk,),
        in_specs=[
            pl.BlockSpec((blk, dhin), lambda i: (i, 0)),
            pl.BlockSpec((blk, dhin), lambda i: (i, 0)),
            pl.BlockSpec((blk, 1), lambda i: (i, 0)),
            pl.BlockSpec((2 * dhin, dout), lambda i: (0, 0)),
        ],
        out_specs=[
            pl.BlockSpec((blk, dh), lambda i: (i, 0)),
            pl.BlockSpec((blk, dh), lambda i: (i, 0)),
        ],
        out_shape=[
            jax.ShapeDtypeStruct((n, dh), jnp.float32),
            jax.ShapeDtypeStruct((n, dh), jnp.float32),
        ],
    )(x0, x1, norm, w)


def _scale_body(x0_ref, x1_ref, norm_ref, o_ref):
    x = jnp.concatenate([x0_ref[...], x1_ref[...]], axis=1)
    o_ref[...] = x * norm_ref[...]


def _final_scale(x0, x1, norm):
    n, dh = x0.shape
    blk = 1000
    return pl.pallas_call(
        _scale_body,
        grid=(n // blk,),
        in_specs=[
            pl.BlockSpec((blk, dh), lambda i: (i, 0)),
            pl.BlockSpec((blk, dh), lambda i: (i, 0)),
            pl.BlockSpec((blk, 1), lambda i: (i, 0)),
        ],
        out_specs=pl.BlockSpec((blk, 2 * dh), lambda i: (i, 0)),
        out_shape=jax.ShapeDtypeStruct((n, 2 * dh), jnp.float32),
    )(x0, x1, norm)


def _make_sc_agg(d2):
    K = 80
    ept = _E // 16
    nch = ept // K
    rpt = 624

    mesh = plsc.VectorSubcoreMesh(core_axis_name="c", subcore_axis_name="s")

    @functools.partial(
        pl.kernel,
        mesh=mesh,
        compiler_params=pltpu.CompilerParams(use_tc_tiling_on_sc=(d2 % 128 == 0)),
        out_type=[
            jax.ShapeDtypeStruct((_N, d2), jnp.float32),
            jax.ShapeDtypeStruct((_N, d2), jnp.float32),
        ],
        scratch_types=[
            pltpu.VMEM((K,), jnp.int32),
            pltpu.VMEM((K,), jnp.int32),
            pltpu.VMEM((K, d2), jnp.float32),
            pltpu.VMEM_SHARED((_N, d2), jnp.float32),
            pltpu.SemaphoreType.DMA,
        ],
    )
    def agg(g0_hbm, g1_hbm, src_hbm, dst_hbm, zero_hbm, out0_hbm, out1_hbm,
            src_v, dst_v, rows_v, acc_sh, sem):
        c = lax.axis_index("c")
        s = lax.axis_index("s")
        row0 = s * rpt
        tail = 16 * rpt

        pltpu.sync_copy(zero_hbm, acc_sh.at[pl.ds(row0, rpt)])

        @pl.when(s == 15)
        def _():
            pltpu.sync_copy(zero_hbm.at[pl.ds(0, 16)],
                            acc_sh.at[pl.ds(tail, 16)])

        plsc.subcore_barrier()

        def run(g_hbm, out_hbm):
            ebase = s * ept

            def body(g, carry):
                base = ebase + g * K
                pltpu.sync_copy(src_hbm.at[pl.ds(base, K)], src_v)
                pltpu.sync_copy(dst_hbm.at[pl.ds(base, K)], dst_v)
                pltpu.async_copy(g_hbm.at[src_v], rows_v, sem).wait()
                pltpu.sync_copy(rows_v, acc_sh.at[dst_v], add=True)
                return carry

            lax.fori_loop(0, nch, body, 0)
            plsc.subcore_barrier()
            pltpu.sync_copy(acc_sh.at[pl.ds(row0, rpt)],
                            out_hbm.at[pl.ds(row0, rpt)])

            @pl.when(s == 15)
            def _():
                pltpu.sync_copy(acc_sh.at[pl.ds(tail, 16)],
                                out_hbm.at[pl.ds(tail, 16)])

        @pl.when(c == 0)
        def _():
            run(g0_hbm, out0_hbm)

        @pl.when(c == 1)
        def _():
            run(g1_hbm, out1_hbm)

    return agg


_sc_agg_128 = _make_sc_agg(128)
_sc_agg_32 = _make_sc_agg(32)


def kernel(features, norm, edge_index, W0, W1, W2):
    src = edge_index[0]
    dst = edge_index[1]
    z128 = jnp.zeros((624, 128), jnp.float32)
    z32 = jnp.zeros((624, 32), jnp.float32)
    f0 = features[:, :128]
    f1 = features[:, 128:]

    g0, g1 = _tc_layer(f0, f1, norm, W0, relu_in=False)
    h0, h1 = _sc_agg_128(g0, g1, src, dst, z128)
    g0, g1 = _tc_layer(h0, h1, norm, W1, relu_in=True)
    h0, h1 = _sc_agg_128(g0, g1, src, dst, z128)
    g0, g1 = _tc_layer(h0, h1, norm, W2, relu_in=True)
    h0, h1 = _sc_agg_32(g0, g1, src, dst, z32)
    return _final_scale(h0, h1, norm)


# speedup vs baseline: 7.1158x; 2.1390x over previous
import functools

import jax
import jax.numpy as jnp
from jax import lax
from jax.experimental import pallas as pl
from jax.experimental.pallas import tpu as pltpu
from jax.experimental.pallas import tpu_sc as plsc

_N = 10000
_E = 160000


def _tc_layer_body(x0_ref, x1_ref, norm_ref, w_ref, out0_ref, out1_ref,
                   *, relu_in, dh):
    x = jnp.concatenate([x0_ref[...], x1_ref[...]], axis=1)
    nrm = norm_ref[...]
    if relu_in:
        x = jnp.maximum(x * nrm, 0.0)
    y = jnp.dot(x, w_ref[...], preferred_element_type=jnp.float32)
    y = y * nrm
    out0_ref[...] = y[:, :dh]
    out1_ref[...] = y[:, dh:]


def _tc_layer(x0, x1, norm, w, relu_in):
    n, dhin = x0.shape
    dout = w.shape[1]
    dh = dout // 2
    blk = 1000
    return pl.pallas_call(
        functools.partial(_tc_layer_body, relu_in=relu_in, dh=dh),
        grid=(n // blk,),
        in_specs=[
            pl.BlockSpec((blk, dhin), lambda i: (i, 0)),
            pl.BlockSpec((blk, dhin), lambda i: (i, 0)),
            pl.BlockSpec((blk, 1), lambda i: (i, 0)),
            pl.BlockSpec((2 * dhin, dout), lambda i: (0, 0)),
        ],
        out_specs=[
            pl.BlockSpec((blk, dh), lambda i: (i, 0)),
            pl.BlockSpec((blk, dh), lambda i: (i, 0)),
        ],
        out_shape=[
            jax.ShapeDtypeStruct((n, dh), jnp.float32),
            jax.ShapeDtypeStruct((n, dh), jnp.float32),
        ],
    )(x0, x1, norm, w)


def _scale_body(x0_ref, x1_ref, norm_ref, o_ref):
    x = jnp.concatenate([x0_ref[...], x1_ref[...]], axis=1)
    o_ref[...] = x * norm_ref[...]


def _final_scale(x0, x1, norm):
    n, dh = x0.shape
    blk = 1000
    return pl.pallas_call(
        _scale_body,
        grid=(n // blk,),
        in_specs=[
            pl.BlockSpec((blk, dh), lambda i: (i, 0)),
            pl.BlockSpec((blk, dh), lambda i: (i, 0)),
            pl.BlockSpec((blk, 1), lambda i: (i, 0)),
        ],
        out_specs=pl.BlockSpec((blk, 2 * dh), lambda i: (i, 0)),
        out_shape=jax.ShapeDtypeStruct((n, 2 * dh), jnp.float32),
    )(x0, x1, norm)


def _make_sc_agg(d2):
    K = 80
    ept = _E // 16
    nch = ept // K
    rpt = 624

    mesh = plsc.VectorSubcoreMesh(core_axis_name="c", subcore_axis_name="s")

    @functools.partial(
        pl.kernel,
        mesh=mesh,
        compiler_params=pltpu.CompilerParams(use_tc_tiling_on_sc=(d2 % 128 == 0)),
        out_type=[
            jax.ShapeDtypeStruct((_N, d2), jnp.float32),
            jax.ShapeDtypeStruct((_N, d2), jnp.float32),
        ],
        scratch_types=[
            pltpu.VMEM((25, K), jnp.int32),
            pltpu.VMEM((25, K), jnp.int32),
            pltpu.VMEM((2, K, d2), jnp.float32),
            pltpu.VMEM_SHARED((_N, d2), jnp.float32),
            pltpu.SemaphoreType.DMA,
            pltpu.SemaphoreType.DMA,
        ],
    )
    def agg(g0_hbm, g1_hbm, src_hbm, dst_hbm, zero_hbm, out0_hbm, out1_hbm,
            src_v, dst_v, rows_v, acc_sh, sem0, sem1):
        c = lax.axis_index("c")
        s = lax.axis_index("s")
        row0 = s * rpt
        tail = 16 * rpt

        pltpu.sync_copy(zero_hbm, acc_sh.at[pl.ds(row0, rpt)])

        @pl.when(s == 15)
        def _():
            pltpu.sync_copy(zero_hbm.at[pl.ds(0, 16)],
                            acc_sh.at[pl.ds(tail, 16)])

        plsc.subcore_barrier()

        def run(g_hbm, out_hbm):
            sems = (sem0, sem1)

            def gather(j, b):
                pltpu.async_copy(g_hbm.at[src_v.at[j]], rows_v.at[b], sems[b])

            def gwait(j, b):
                pltpu.make_async_copy(
                    g_hbm.at[src_v.at[j]], rows_v.at[b], sems[b]).wait()

            def scat(j, b):
                pltpu.sync_copy(rows_v.at[b], acc_sh.at[dst_v.at[j]], add=True)

            for blk in range(nch // 25):
                pltpu.sync_copy(src_hbm.at[s, blk], src_v)
                pltpu.sync_copy(dst_hbm.at[s, blk], dst_v)
                gather(0, 0)

                def body(i, carry):
                    j0 = 2 * i
                    gather(j0 + 1, 1)
                    gwait(j0, 0)
                    scat(j0, 0)
                    gather(j0 + 2, 0)
                    gwait(j0 + 1, 1)
                    scat(j0 + 1, 1)
                    return carry

                lax.fori_loop(0, 12, body, 0)
                gwait(24, 0)
                scat(24, 0)

            plsc.subcore_barrier()
            pltpu.sync_copy(acc_sh.at[pl.ds(row0, rpt)],
                            out_hbm.at[pl.ds(row0, rpt)])

            @pl.when(s == 15)
            def _():
                pltpu.sync_copy(acc_sh.at[pl.ds(tail, 16)],
                                out_hbm.at[pl.ds(tail, 16)])

        @pl.when(c == 0)
        def _():
            run(g0_hbm, out0_hbm)

        @pl.when(c == 1)
        def _():
            run(g1_hbm, out1_hbm)

    return agg


_sc_agg_128 = _make_sc_agg(128)
_sc_agg_32 = _make_sc_agg(32)


def kernel(features, norm, edge_index, W0, W1, W2):
    src = edge_index[0].reshape(16, 5, 25, 80)
    dst = edge_index[1].reshape(16, 5, 25, 80)
    z128 = jnp.zeros((624, 128), jnp.float32)
    z32 = jnp.zeros((624, 32), jnp.float32)
    f0 = features[:, :128]
    f1 = features[:, 128:]

    g0, g1 = _tc_layer(f0, f1, norm, W0, relu_in=False)
    h0, h1 = _sc_agg_128(g0, g1, src, dst, z128)
    g0, g1 = _tc_layer(h0, h1, norm, W1, relu_in=True)
    h0, h1 = _sc_agg_128(g0, g1, src, dst, z128)
    g0, g1 = _tc_layer(h0, h1, norm, W2, relu_in=True)
    h0, h1 = _sc_agg_32(g0, g1, src, dst, z32)
    return _final_scale(h0, h1, norm)


# speedup vs baseline: 8.1290x; 1.1424x over previous
import functools

import jax
import jax.numpy as jnp
from jax import lax
from jax.experimental import pallas as pl
from jax.experimental.pallas import tpu as pltpu
from jax.experimental.pallas import tpu_sc as plsc

_N = 10000
_E = 160000


def _tc_layer_body(*refs, relu_in, dh, nx):
    x_refs = refs[:nx]
    norm_ref, w_ref, out0_ref, out1_ref = refs[nx:]
    if nx == 1:
        x = x_refs[0][...]
    else:
        x = jnp.concatenate([r[...] for r in x_refs], axis=1)
    nrm = norm_ref[...]
    if relu_in:
        x = jnp.maximum(x * nrm, 0.0)
    y = jnp.dot(x, w_ref[...], preferred_element_type=jnp.float32)
    y = y * nrm
    out0_ref[...] = y[:, :dh]
    out1_ref[...] = y[:, dh:]


def _tc_layer(xs, norm, w, relu_in):
    n = xs[0].shape[0]
    dout = w.shape[1]
    dh = dout // 2
    blk = 2000
    return pl.pallas_call(
        functools.partial(_tc_layer_body, relu_in=relu_in, dh=dh, nx=len(xs)),
        grid=(n // blk,),
        in_specs=[
            pl.BlockSpec((blk, x.shape[1]), lambda i: (i, 0)) for x in xs
        ] + [
            pl.BlockSpec((blk, 1), lambda i: (i, 0)),
            pl.BlockSpec(w.shape, lambda i: (0, 0)),
        ],
        out_specs=[
            pl.BlockSpec((blk, dh), lambda i: (i, 0)),
            pl.BlockSpec((blk, dh), lambda i: (i, 0)),
        ],
        out_shape=[
            jax.ShapeDtypeStruct((n, dh), jnp.float32),
            jax.ShapeDtypeStruct((n, dh), jnp.float32),
        ],
    )(*xs, norm, w)


def _scale_body(x0_ref, x1_ref, norm_ref, o_ref):
    x = jnp.concatenate([x0_ref[...], x1_ref[...]], axis=1)
    o_ref[...] = x * norm_ref[...]


def _final_scale(x0, x1, norm):
    n, dh = x0.shape
    blk = 2000
    return pl.pallas_call(
        _scale_body,
        grid=(n // blk,),
        in_specs=[
            pl.BlockSpec((blk, dh), lambda i: (i, 0)),
            pl.BlockSpec((blk, dh), lambda i: (i, 0)),
            pl.BlockSpec((blk, 1), lambda i: (i, 0)),
        ],
        out_specs=pl.BlockSpec((blk, 2 * dh), lambda i: (i, 0)),
        out_shape=jax.ShapeDtypeStruct((n, 2 * dh), jnp.float32),
    )(x0, x1, norm)


_K = 125
_NSTG = 4
_CPS = 20


def _make_sc_agg(d2):
    rpt = 624

    mesh = plsc.VectorSubcoreMesh(core_axis_name="c", subcore_axis_name="s")

    @functools.partial(
        pl.kernel,
        mesh=mesh,
        compiler_params=pltpu.CompilerParams(use_tc_tiling_on_sc=(d2 % 128 == 0)),
        out_type=[
            jax.ShapeDtypeStruct((_N, d2), jnp.float32),
            jax.ShapeDtypeStruct((_N, d2), jnp.float32),
        ],
        scratch_types=[
            pltpu.VMEM((_CPS, _K), jnp.int32),
            pltpu.VMEM((_CPS, _K), jnp.int32),
            pltpu.VMEM((2, _K, d2), jnp.float32),
            pltpu.VMEM_SHARED((_N, d2), jnp.float32),
            pltpu.SemaphoreType.DMA,
            pltpu.SemaphoreType.DMA,
        ],
    )
    def agg(g0_hbm, g1_hbm, src_hbm, dst_hbm, zero_hbm, out0_hbm, out1_hbm,
            src_v, dst_v, rows_v, acc_sh, sem0, sem1):
        c = lax.axis_index("c")
        s = lax.axis_index("s")
        row0 = s * rpt
        tail = 16 * rpt

        pltpu.sync_copy(zero_hbm, acc_sh.at[pl.ds(row0, rpt)])

        @pl.when(s == 15)
        def _():
            pltpu.sync_copy(zero_hbm.at[pl.ds(0, 16)],
                            acc_sh.at[pl.ds(tail, 16)])

        plsc.subcore_barrier()

        def run(g_hbm, out_hbm):
            sems = (sem0, sem1)

            def gather(j, b):
                pltpu.async_copy(g_hbm.at[src_v.at[j]], rows_v.at[b], sems[b])

            def gwait(j, b):
                pltpu.make_async_copy(
                    g_hbm.at[src_v.at[j]], rows_v.at[b], sems[b]).wait()

            def scat(j, b):
                pltpu.sync_copy(rows_v.at[b], acc_sh.at[dst_v.at[j]], add=True)

            for blk in range(_NSTG):
                pltpu.sync_copy(src_hbm.at[s, blk], src_v)
                pltpu.sync_copy(dst_hbm.at[s, blk], dst_v)
                gather(0, 0)

                def body(i, carry):
                    j0 = 2 * i
                    gather(j0 + 1, 1)
                    gwait(j0, 0)
                    scat(j0, 0)

                    @pl.when(i < _CPS // 2 - 1)
                    def _():
                        gather(j0 + 2, 0)

                    gwait(j0 + 1, 1)
                    scat(j0 + 1, 1)
                    return carry

                lax.fori_loop(0, _CPS // 2, body, 0)

            plsc.subcore_barrier()
            pltpu.sync_copy(acc_sh.at[pl.ds(row0, rpt)],
                            out_hbm.at[pl.ds(row0, rpt)])

            @pl.when(s == 15)
            def _():
                pltpu.sync_copy(acc_sh.at[pl.ds(tail, 16)],
                                out_hbm.at[pl.ds(tail, 16)])

        @pl.when(c == 0)
        def _():
            run(g0_hbm, out0_hbm)

        @pl.when(c == 1)
        def _():
            run(g1_hbm, out1_hbm)

    return agg


_sc_agg_128 = _make_sc_agg(128)
_sc_agg_32 = _make_sc_agg(32)


def kernel(features, norm, edge_index, W0, W1, W2):
    src = edge_index[0].reshape(16, _NSTG, _CPS, _K)
    dst = edge_index[1].reshape(16, _NSTG, _CPS, _K)
    z128 = jnp.zeros((624, 128), jnp.float32)
    z32 = jnp.zeros((624, 32), jnp.float32)

    g0, g1 = _tc_layer([features], norm, W0, relu_in=False)
    h0, h1 = _sc_agg_128(g0, g1, src, dst, z128)
    g0, g1 = _tc_layer([h0, h1], norm, W1, relu_in=True)
    h0, h1 = _sc_agg_128(g0, g1, src, dst, z128)
    g0, g1 = _tc_layer([h0, h1], norm, W2, relu_in=True)
    h0, h1 = _sc_agg_32(g0, g1, src, dst, z32)
    return _final_scale(h0, h1, norm)
